# Initial kernel scaffold; baseline (speedup 1.0000x reference)
import functools

import jax
import jax.numpy as jnp
from jax import lax
from jax.experimental import pallas as pl
from jax.experimental.pallas import tpu as pltpu
from jax.experimental.pallas import tpu_sc as plsc

_F32 = jnp.float32


def _proj_body(x_ref, wv_ref, wsc_ref, xo_ref, so_ref):
    xb = x_ref[...]
    xo_ref[...] = jnp.dot(xb, wv_ref[...], preferred_element_type=_F32)
    so_ref[...] = jnp.dot(xb, wsc_ref[...], preferred_element_type=_F32)


def _project(Xf, value, wsc, blk):
    rows = Xf.shape[0]
    C = Xf.shape[1]
    grid = rows // blk
    return pl.pallas_call(
        _proj_body,
        grid=(grid,),
        in_specs=[
            pl.BlockSpec((blk, C), lambda i: (i, 0)),
            pl.BlockSpec((C, C), lambda i: (0, 0)),
            pl.BlockSpec((C, 8), lambda i: (0, 0)),
        ],
        out_specs=[
            pl.BlockSpec((blk, C), lambda i: (i, 0)),
            pl.BlockSpec((blk, 8), lambda i: (i, 0)),
        ],
        out_shape=[
            jax.ShapeDtypeStruct((rows, C), _F32),
            jax.ShapeDtypeStruct((rows, 8), _F32),
        ],
    )(Xf, value, wsc)


def _update_body(x_ref, a_ref, w1_ref, w2_ref, cb_ref, g_ref, b_ref, o_ref):
    xb = x_ref[...]
    h = (jnp.dot(xb, w1_ref[...], preferred_element_type=_F32)
         + jnp.dot(a_ref[...], w2_ref[...], preferred_element_type=_F32)
         + cb_ref[...])
    mu = jnp.mean(h, axis=-1, keepdims=True)
    var = jnp.mean((h - mu) * (h - mu), axis=-1, keepdims=True)
    hn = (h - mu) * lax.rsqrt(var + 1e-5) * g_ref[...] + b_ref[...]
    o_ref[...] = xb + jnp.maximum(hn, 0.0)


def _update(xf, aggr, w1, w2, cat_b, ln_g, ln_b, blk):
    rows, C = xf.shape
    grid = rows // blk
    vec = lambda i: (0, 0)
    return pl.pallas_call(
        _update_body,
        grid=(grid,),
        in_specs=[
            pl.BlockSpec((blk, C), lambda i: (i, 0)),
            pl.BlockSpec((blk, C), lambda i: (i, 0)),
            pl.BlockSpec((C, C), vec),
            pl.BlockSpec((C, C), vec),
            pl.BlockSpec((1, C), vec),
            pl.BlockSpec((1, C), vec),
            pl.BlockSpec((1, C), vec),
        ],
        out_specs=pl.BlockSpec((blk, C), lambda i: (i, 0)),
        out_shape=jax.ShapeDtypeStruct((rows, C), _F32),
    )(xf, aggr, w1, w2, cat_b, ln_g, ln_b)


def _make_sc_aggregate(N, C, E_pad, K, n_tiles):
    chunks = E_pad // (n_tiles * K)
    rows_per_tile = N // n_tiles
    wb = 125
    n_wb = rows_per_tile // wb
    mesh = plsc.VectorSubcoreMesh(core_axis_name="c", subcore_axis_name="s")

    @functools.partial(
        pl.kernel,
        mesh=mesh,
        out_type=jax.ShapeDtypeStruct((2 * N, C), _F32),
        scratch_types=[
            pltpu.VMEM((N,), _F32),
            pltpu.VMEM((N,), _F32),
            pltpu.VMEM((C,), _F32),
            pltpu.VMEM((chunks, K), jnp.int32),
            pltpu.VMEM((chunks, K), jnp.int32),
            pltpu.VMEM((chunks, K), _F32),
            pltpu.VMEM((chunks, K), _F32),
            pltpu.VMEM((K,), jnp.int32),
            pltpu.VMEM((K,), _F32),
            pltpu.VMEM((K, C), _F32),
            pltpu.VMEM_SHARED((N, C), _F32),
            pltpu.SemaphoreType.DMA,
        ],
    )
    def sc_aggregate(x_hbm, asrc_hbm, adst_hbm, src_hbm, dst_hbm, ew_hbm,
                     ewe_hbm, we_hbm, out_hbm,
                     asrc_v, adst_v, we_v, src_v, dst_v, ew_v, ewe_v,
                     soff_v, att_v, rows_v, agg_sh, sem):
        c = lax.axis_index("c")
        s = lax.axis_index("s")
        cN = c * N

        pltpu.sync_copy(asrc_hbm.at[pl.ds(cN, N)], asrc_v)
        pltpu.sync_copy(adst_hbm.at[pl.ds(cN, N)], adst_v)
        pltpu.sync_copy(we_hbm, we_v)
        t0 = s * chunks
        pltpu.sync_copy(src_hbm.at[pl.ds(t0, chunks)], src_v)
        pltpu.sync_copy(dst_hbm.at[pl.ds(t0, chunks)], dst_v)
        pltpu.sync_copy(ew_hbm.at[pl.ds(t0, chunks)], ew_v)
        pltpu.sync_copy(ewe_hbm.at[pl.ds(t0, chunks)], ewe_v)

        def _zrow(e, _):
            for j in range(C // 16):
                rows_v[e, pl.ds(j * 16, 16)] = jnp.zeros((16,), _F32)
            return 0
        lax.fori_loop(0, K, _zrow, 0)
        for r in range(n_wb):
            pltpu.sync_copy(rows_v.at[pl.ds(0, wb)],
                            agg_sh.at[pl.ds(s * rows_per_tile + r * wb, wb)])
        plsc.subcore_barrier()

        def _chunk(i, _):
            for g in range(K // 16):
                sl = pl.ds(g * 16, 16)
                isrc = src_v[i, sl]
                idst = dst_v[i, sl]
                a_s = plsc.load_gather(asrc_v, [isrc])
                a_d = plsc.load_gather(adst_v, [idst])
                z = a_s + a_d + ewe_v[i, sl]
                att_v[sl] = 1.0 / (1.0 + jnp.exp(-z))
                soff_v[sl] = isrc + cN
            pltpu.async_copy(x_hbm.at[soff_v], rows_v, sem).wait()

            def _edge(e, _):
                att_e = att_v[e]
                ew_e = ew_v[i, e]
                for j in range(C // 16):
                    slj = pl.ds(j * 16, 16)
                    t = ew_e * we_v[slj]
                    m = att_e / (1.0 + jnp.exp(-t))
                    rows_v[e, slj] = m * rows_v[e, slj]
                return 0
            lax.fori_loop(0, K, _edge, 0)

            pltpu.sync_copy(rows_v, agg_sh.at[dst_v.at[i]], add=True)
            return 0
        lax.fori_loop(0, chunks, _chunk, 0)

        plsc.subcore_barrier()
        for r in range(n_wb):
            r0 = s * rows_per_tile + r * wb
            pltpu.sync_copy(agg_sh.at[pl.ds(r0, wb)], rows_v.at[pl.ds(0, wb)])
            pltpu.sync_copy(rows_v.at[pl.ds(0, wb)], out_hbm.at[pl.ds(cN + r0, wb)])

    return sc_aggregate


def kernel(X, edge_index, edge_weight, value, key_w, query_w, weight_e,
           att_W, att_b, cat_W, cat_b, ln_g, ln_b):
    B, N, C = X.shape
    E = edge_index.shape[1]
    n_tiles = 16
    K = 128
    per_tile = -(-E // n_tiles)
    per_tile = -(-per_tile // K) * K
    E_pad = per_tile * n_tiles

    vk = value @ (key_w @ att_W[:C, 0])
    vq = value @ (query_w @ att_W[C:2 * C, 0])
    ce = weight_e[0] @ att_W[2 * C:, 0]
    wsc = jnp.zeros((C, 8), _F32).at[:, 0].set(vq).at[:, 1].set(vk)

    src = edge_index[0].astype(jnp.int32)
    dst = edge_index[1].astype(jnp.int32)
    ew = edge_weight.astype(_F32)
    ew_eff = ew * ce + att_b[0]
    pad = E_pad - E
    if pad:
        iz = jnp.zeros((pad,), jnp.int32)
        src = jnp.concatenate([src, iz])
        dst = jnp.concatenate([dst, iz])
        ew = jnp.concatenate([ew, jnp.zeros((pad,), _F32)])
        ew_eff = jnp.concatenate([ew_eff, jnp.full((pad,), -1e30, _F32)])
    chunks = per_tile // K
    src2 = src.reshape(n_tiles * chunks, K)
    dst2 = dst.reshape(n_tiles * chunks, K)
    ew2 = ew.reshape(n_tiles * chunks, K)
    ewe2 = ew_eff.reshape(n_tiles * chunks, K)

    Xf = X.reshape(B * N, C)
    xf, sc2 = _project(Xf, value, wsc, blk=1000)
    asrc = sc2[:, 0]
    adst = sc2[:, 1]

    sc_fn = _make_sc_aggregate(N, C, E_pad, K, n_tiles)
    aggr = sc_fn(xf, asrc, adst, src2, dst2, ew2, ewe2, weight_e[0])

    out = _update(xf, aggr, cat_W[:C], cat_W[C:], cat_b.reshape(1, C),
                  ln_g.reshape(1, C), ln_b.reshape(1, C), blk=1000)
    return out.reshape(B, N, C)


# speedup vs baseline: 5.0255x; 5.0255x over previous
import functools

import jax
import jax.numpy as jnp
from jax import lax
from jax.experimental import pallas as pl
from jax.experimental.pallas import tpu as pltpu
from jax.experimental.pallas import tpu_sc as plsc

_F32 = jnp.float32


def _proj_body(x_ref, wv_ref, wsc_ref, xo_ref, so_ref):
    xb = x_ref[...]
    xo_ref[...] = jnp.dot(xb, wv_ref[...], preferred_element_type=_F32)
    so_ref[...] = jnp.dot(xb, wsc_ref[...], preferred_element_type=_F32)


def _project(Xf, value, wsc, blk):
    rows = Xf.shape[0]
    C = Xf.shape[1]
    grid = rows // blk
    return pl.pallas_call(
        _proj_body,
        grid=(grid,),
        in_specs=[
            pl.BlockSpec((blk, C), lambda i: (i, 0)),
            pl.BlockSpec((C, C), lambda i: (0, 0)),
            pl.BlockSpec((C, 8), lambda i: (0, 0)),
        ],
        out_specs=[
            pl.BlockSpec((blk, C), lambda i: (i, 0)),
            pl.BlockSpec((blk, 8), lambda i: (i, 0)),
        ],
        out_shape=[
            jax.ShapeDtypeStruct((rows, C), _F32),
            jax.ShapeDtypeStruct((rows, 8), _F32),
        ],
    )(Xf, value, wsc)


def _update_body(x_ref, a_ref, w1_ref, w2_ref, cb_ref, g_ref, b_ref, o_ref):
    xb = x_ref[...]
    h = (jnp.dot(xb, w1_ref[...], preferred_element_type=_F32)
         + jnp.dot(a_ref[...], w2_ref[...], preferred_element_type=_F32)
         + cb_ref[...])
    mu = jnp.mean(h, axis=-1, keepdims=True)
    var = jnp.mean((h - mu) * (h - mu), axis=-1, keepdims=True)
    hn = (h - mu) * lax.rsqrt(var + 1e-5) * g_ref[...] + b_ref[...]
    o_ref[...] = xb + jnp.maximum(hn, 0.0)


def _update(xf, aggr, w1, w2, cat_b, ln_g, ln_b, blk):
    rows, C = xf.shape
    grid = rows // blk
    vec = lambda i: (0, 0)
    return pl.pallas_call(
        _update_body,
        grid=(grid,),
        in_specs=[
            pl.BlockSpec((blk, C), lambda i: (i, 0)),
            pl.BlockSpec((blk, C), lambda i: (i, 0)),
            pl.BlockSpec((C, C), vec),
            pl.BlockSpec((C, C), vec),
            pl.BlockSpec((1, C), vec),
            pl.BlockSpec((1, C), vec),
            pl.BlockSpec((1, C), vec),
        ],
        out_specs=pl.BlockSpec((blk, C), lambda i: (i, 0)),
        out_shape=jax.ShapeDtypeStruct((rows, C), _F32),
    )(xf, aggr, w1, w2, cat_b, ln_g, ln_b)


def _make_sc_aggregate(N, C, E_pad, K, n_tiles):
    chunks = E_pad // (n_tiles * K)
    wb = 80
    nburst = N // wb
    max_b = -(-nburst // n_tiles)
    mesh = plsc.VectorSubcoreMesh(core_axis_name="c", subcore_axis_name="s")

    @functools.partial(
        pl.kernel,
        mesh=mesh,
        compiler_params=pltpu.CompilerParams(needs_layout_passes=False),
        out_type=jax.ShapeDtypeStruct((2 * N, C), _F32),
        scratch_types=[
            pltpu.VMEM((N,), _F32),
            pltpu.VMEM((N,), _F32),
            pltpu.VMEM((C,), _F32),
            pltpu.VMEM((K,), jnp.int32),
            pltpu.VMEM((K,), jnp.int32),
            pltpu.VMEM((K,), _F32),
            pltpu.VMEM((K,), _F32),
            pltpu.VMEM((K,), jnp.int32),
            pltpu.VMEM((K,), _F32),
            pltpu.VMEM((K, C), _F32),
            pltpu.VMEM_SHARED((N, C), _F32),
            pltpu.SemaphoreType.DMA,
        ],
    )
    def sc_aggregate(x_hbm, asrc_hbm, adst_hbm, src_hbm, dst_hbm, ew_hbm,
                     ewe_hbm, we_hbm, out_hbm,
                     asrc_v, adst_v, we_v, src_v, dst_v, ew_v, ewe_v,
                     soff_v, att_v, rows_v, agg_sh, sem):
        c = lax.axis_index("c")
        s = lax.axis_index("s")
        cN = c * N

        pltpu.sync_copy(asrc_hbm.at[pl.ds(cN, N)], asrc_v)
        pltpu.sync_copy(adst_hbm.at[pl.ds(cN, N)], adst_v)
        pltpu.sync_copy(we_hbm, we_v)
        e0 = s * chunks * K

        def _zrow(e, _):
            for j in range(C // 16):
                rows_v[e, pl.ds(j * 16, 16)] = jnp.zeros((16,), _F32)
            return 0
        lax.fori_loop(0, wb, _zrow, 0)
        for r in range(max_b):
            b = s + r * n_tiles

            @pl.when(b < nburst)
            def _init_burst(b=b):
                pltpu.sync_copy(rows_v.at[pl.ds(0, wb)],
                                agg_sh.at[pl.ds(b * wb, wb)])
        plsc.subcore_barrier()

        def _chunk(i, _):
            base_e = e0 + i * K
            pltpu.sync_copy(src_hbm.at[pl.ds(base_e, K)], src_v)
            pltpu.sync_copy(dst_hbm.at[pl.ds(base_e, K)], dst_v)
            pltpu.sync_copy(ew_hbm.at[pl.ds(base_e, K)], ew_v)
            pltpu.sync_copy(ewe_hbm.at[pl.ds(base_e, K)], ewe_v)
            for g in range(K // 16):
                sl = pl.ds(g * 16, 16)
                isrc = src_v[sl]
                idst = dst_v[sl]
                a_s = plsc.load_gather(asrc_v, [isrc])
                a_d = plsc.load_gather(adst_v, [idst])
                z = a_s + a_d + ewe_v[sl]
                att_v[sl] = 1.0 / (1.0 + jnp.exp(-z))
                soff_v[sl] = isrc + cN
            pltpu.async_copy(x_hbm.at[soff_v], rows_v, sem).wait()

            def _group(g, _):
                sl = pl.ds(g * 16, 16)
                att16 = att_v[sl]
                ew16 = ew_v[sl]
                base = g * 16
                for l in range(16):
                    att_e = att16[l]
                    ew_e = ew16[l]
                    for j in range(C // 16):
                        slj = pl.ds(j * 16, 16)
                        t = ew_e * we_v[slj]
                        m = att_e / (1.0 + jnp.exp(-t))
                        rows_v[base + l, slj] = m * rows_v[base + l, slj]
                return 0
            lax.fori_loop(0, K // 16, _group, 0)

            pltpu.sync_copy(rows_v, agg_sh.at[dst_v], add=True)
            return 0
        lax.fori_loop(0, chunks, _chunk, 0)

        plsc.subcore_barrier()
        for r in range(max_b):
            b = s + r * n_tiles

            @pl.when(b < nburst)
            def _write_burst(b=b):
                pltpu.sync_copy(agg_sh.at[pl.ds(b * wb, wb)],
                                rows_v.at[pl.ds(0, wb)])
                pltpu.sync_copy(rows_v.at[pl.ds(0, wb)],
                                out_hbm.at[pl.ds(cN + b * wb, wb)])

    return sc_aggregate


def kernel(X, edge_index, edge_weight, value, key_w, query_w, weight_e,
           att_W, att_b, cat_W, cat_b, ln_g, ln_b):
    B, N, C = X.shape
    E = edge_index.shape[1]
    n_tiles = 16
    K = 128
    per_tile = -(-E // n_tiles)
    per_tile = -(-per_tile // (8 * K)) * (8 * K)
    E_pad = per_tile * n_tiles

    vk = value @ (key_w @ att_W[:C, 0])
    vq = value @ (query_w @ att_W[C:2 * C, 0])
    ce = weight_e[0] @ att_W[2 * C:, 0]
    wsc = jnp.zeros((C, 8), _F32).at[:, 0].set(vq).at[:, 1].set(vk)

    src = edge_index[0].astype(jnp.int32)
    dst = edge_index[1].astype(jnp.int32)
    ew = edge_weight.astype(_F32)
    ew_eff = ew * ce + att_b[0]
    pad = E_pad - E
    if pad:
        iz = jnp.zeros((pad,), jnp.int32)
        src = jnp.concatenate([src, iz])
        dst = jnp.concatenate([dst, iz])
        ew = jnp.concatenate([ew, jnp.zeros((pad,), _F32)])
        ew_eff = jnp.concatenate([ew_eff, jnp.full((pad,), -1e30, _F32)])
    Xf = X.reshape(B * N, C)
    xf, sc2 = _project(Xf, value, wsc, blk=1000)
    asrc = sc2[:, 0]
    adst = sc2[:, 1]

    sc_fn = _make_sc_aggregate(N, C, E_pad, K, n_tiles)
    aggr = sc_fn(xf, asrc, adst, src, dst, ew, ew_eff, weight_e[0])

    out = _update(xf, aggr, cat_W[:C], cat_W[C:], cat_b.reshape(1, C),
                  ln_g.reshape(1, C), ln_b.reshape(1, C), blk=1000)
    return out.reshape(B, N, C)


# speedup vs baseline: 16.3405x; 3.2515x over previous
import functools

import jax
import jax.numpy as jnp
from jax import lax
from jax.experimental import pallas as pl
from jax.experimental.pallas import tpu as pltpu
from jax.experimental.pallas import tpu_sc as plsc

_F32 = jnp.float32


def _proj_body(x_ref, wv_ref, wsc_ref, xo_ref, so_ref):
    xb = x_ref[...]
    xo_ref[...] = jnp.dot(xb, wv_ref[...], preferred_element_type=_F32)
    so_ref[...] = jnp.dot(xb, wsc_ref[...], preferred_element_type=_F32)


def _project(Xf, value, wsc, blk):
    rows = Xf.shape[0]
    C = Xf.shape[1]
    grid = rows // blk
    return pl.pallas_call(
        _proj_body,
        grid=(grid,),
        in_specs=[
            pl.BlockSpec((blk, C), lambda i: (i, 0)),
            pl.BlockSpec((C, C), lambda i: (0, 0)),
            pl.BlockSpec((C, 8), lambda i: (0, 0)),
        ],
        out_specs=[
            pl.BlockSpec((blk, C), lambda i: (i, 0)),
            pl.BlockSpec((blk, 8), lambda i: (i, 0)),
        ],
        out_shape=[
            jax.ShapeDtypeStruct((rows, C), _F32),
            jax.ShapeDtypeStruct((rows, 8), _F32),
        ],
    )(Xf, value, wsc)


def _gate_body(ew_ref, we_ref, g_ref):
    t = ew_ref[...] * we_ref[...]
    g_ref[...] = 1.0 / (1.0 + jnp.exp(-t))


def _gate(ew_bc, we_row, blk):
    Ep, C = ew_bc.shape
    return pl.pallas_call(
        _gate_body,
        grid=(Ep // blk,),
        in_specs=[
            pl.BlockSpec((blk, C), lambda i: (i, 0)),
            pl.BlockSpec((1, C), lambda i: (0, 0)),
        ],
        out_specs=pl.BlockSpec((blk, C), lambda i: (i, 0)),
        out_shape=jax.ShapeDtypeStruct((Ep, C), _F32),
    )(ew_bc, we_row)


def _update_body(x_ref, a_ref, w1_ref, w2_ref, cb_ref, g_ref, b_ref, o_ref):
    xb = x_ref[...]
    h = (jnp.dot(xb, w1_ref[...], preferred_element_type=_F32)
         + jnp.dot(a_ref[...], w2_ref[...], preferred_element_type=_F32)
         + cb_ref[...])
    mu = jnp.mean(h, axis=-1, keepdims=True)
    var = jnp.mean((h - mu) * (h - mu), axis=-1, keepdims=True)
    hn = (h - mu) * lax.rsqrt(var + 1e-5) * g_ref[...] + b_ref[...]
    o_ref[...] = xb + jnp.maximum(hn, 0.0)


def _update(xf, aggr, w1, w2, cat_b, ln_g, ln_b, blk):
    rows, C = xf.shape
    grid = rows // blk
    vec = lambda i: (0, 0)
    return pl.pallas_call(
        _update_body,
        grid=(grid,),
        in_specs=[
            pl.BlockSpec((blk, C), lambda i: (i, 0)),
            pl.BlockSpec((blk, C), lambda i: (i, 0)),
            pl.BlockSpec((C, C), vec),
            pl.BlockSpec((C, C), vec),
            pl.BlockSpec((1, C), vec),
            pl.BlockSpec((1, C), vec),
            pl.BlockSpec((1, C), vec),
        ],
        out_specs=pl.BlockSpec((blk, C), lambda i: (i, 0)),
        out_shape=jax.ShapeDtypeStruct((rows, C), _F32),
    )(xf, aggr, w1, w2, cat_b, ln_g, ln_b)


def _make_sc_aggregate(N, C, E_pad, K, n_tiles):
    chunks = E_pad // (n_tiles * K)
    wb = 80
    nburst = N // wb
    max_b = -(-nburst // n_tiles)
    mesh = plsc.VectorSubcoreMesh(core_axis_name="c", subcore_axis_name="s")

    @functools.partial(
        pl.kernel,
        mesh=mesh,
        compiler_params=pltpu.CompilerParams(needs_layout_passes=False),
        out_type=jax.ShapeDtypeStruct((2 * N, C), _F32),
        scratch_types=[
            pltpu.VMEM((N,), _F32),
            pltpu.VMEM((N,), _F32),
            pltpu.VMEM((K,), jnp.int32),
            pltpu.VMEM((K,), jnp.int32),
            pltpu.VMEM((K,), _F32),
            pltpu.VMEM((K,), jnp.int32),
            pltpu.VMEM((K, C), _F32),
            pltpu.VMEM((K, C), _F32),
            pltpu.VMEM_SHARED((N, C), _F32),
            pltpu.SemaphoreType.DMA,
        ],
    )
    def sc_aggregate(x_hbm, asrc_hbm, adst_hbm, src_hbm, dst_hbm,
                     ewe_hbm, gate_hbm, out_hbm,
                     asrc_v, adst_v, src_v, dst_v, ewe_v,
                     soff_v, g_v, rows_v, agg_sh, sem):
        c = lax.axis_index("c")
        s = lax.axis_index("s")
        cN = c * N

        pltpu.sync_copy(asrc_hbm.at[pl.ds(cN, N)], asrc_v)
        pltpu.sync_copy(adst_hbm.at[pl.ds(cN, N)], adst_v)
        e0 = s * chunks * K

        def _zrow(e, _):
            for j in range(C // 16):
                rows_v[e, pl.ds(j * 16, 16)] = jnp.zeros((16,), _F32)
            return 0
        lax.fori_loop(0, wb, _zrow, 0)
        for r in range(max_b):
            b = s + r * n_tiles

            @pl.when(b < nburst)
            def _init_burst(b=b):
                pltpu.sync_copy(rows_v.at[pl.ds(0, wb)],
                                agg_sh.at[pl.ds(b * wb, wb)])
        plsc.subcore_barrier()

        def _chunk(i, _):
            base_e = e0 + i * K
            pltpu.sync_copy(src_hbm.at[pl.ds(base_e, K)], src_v)
            pltpu.sync_copy(dst_hbm.at[pl.ds(base_e, K)], dst_v)
            pltpu.sync_copy(ewe_hbm.at[pl.ds(base_e, K)], ewe_v)
            for g in range(K // 16):
                sl = pl.ds(g * 16, 16)
                soff_v[sl] = src_v[sl] + cN
            gather = pltpu.async_copy(x_hbm.at[soff_v], rows_v, sem)
            pltpu.sync_copy(gate_hbm.at[pl.ds(base_e, K)], g_v)
            gather.wait()

            def _group(g, _):
                sl = pl.ds(g * 16, 16)
                isrc = src_v[sl]
                idst = dst_v[sl]
                a_s = plsc.load_gather(asrc_v, [isrc])
                a_d = plsc.load_gather(adst_v, [idst])
                att16 = 1.0 / (1.0 + jnp.exp(-(a_s + a_d + ewe_v[sl])))
                base = g * 16
                for l in range(16):
                    att_e = att16[l]
                    for j in range(C // 16):
                        slj = pl.ds(j * 16, 16)
                        rows_v[base + l, slj] = (att_e * g_v[base + l, slj]
                                                 * rows_v[base + l, slj])
                return 0
            lax.fori_loop(0, K // 16, _group, 0)

            pltpu.sync_copy(rows_v, agg_sh.at[dst_v], add=True)
            return 0
        lax.fori_loop(0, chunks, _chunk, 0)

        plsc.subcore_barrier()
        for r in range(max_b):
            b = s + r * n_tiles

            @pl.when(b < nburst)
            def _write_burst(b=b):
                pltpu.sync_copy(agg_sh.at[pl.ds(b * wb, wb)],
                                rows_v.at[pl.ds(0, wb)])
                pltpu.sync_copy(rows_v.at[pl.ds(0, wb)],
                                out_hbm.at[pl.ds(cN + b * wb, wb)])

    return sc_aggregate


def kernel(X, edge_index, edge_weight, value, key_w, query_w, weight_e,
           att_W, att_b, cat_W, cat_b, ln_g, ln_b):
    B, N, C = X.shape
    E = edge_index.shape[1]
    n_tiles = 16
    K = 96
    per_tile = -(-E // n_tiles)
    per_tile = -(-per_tile // K) * K
    E_pad = per_tile * n_tiles

    vk = value @ (key_w @ att_W[:C, 0])
    vq = value @ (query_w @ att_W[C:2 * C, 0])
    ce = weight_e[0] @ att_W[2 * C:, 0]
    wsc = jnp.zeros((C, 8), _F32).at[:, 0].set(vq).at[:, 1].set(vk)

    src = edge_index[0].astype(jnp.int32)
    dst = edge_index[1].astype(jnp.int32)
    ew = edge_weight.astype(_F32)
    ew_eff = ew * ce + att_b[0]
    pad = E_pad - E
    if pad:
        iz = jnp.zeros((pad,), jnp.int32)
        src = jnp.concatenate([src, iz])
        dst = jnp.concatenate([dst, iz])
        ew = jnp.concatenate([ew, jnp.zeros((pad,), _F32)])
        ew_eff = jnp.concatenate([ew_eff, jnp.full((pad,), -1e30, _F32)])
    Xf = X.reshape(B * N, C)
    xf, sc2 = _project(Xf, value, wsc, blk=1000)
    asrc = sc2[:, 0]
    adst = sc2[:, 1]

    ew_bc = jnp.broadcast_to(ew[:, None], (E_pad, C))
    gate = _gate(ew_bc, weight_e[0].reshape(1, C), blk=1280)

    sc_fn = _make_sc_aggregate(N, C, E_pad, K, n_tiles)
    aggr = sc_fn(xf, asrc, adst, src, dst, ew_eff, gate)

    out = _update(xf, aggr, cat_W[:C], cat_W[C:], cat_b.reshape(1, C),
                  ln_g.reshape(1, C), ln_b.reshape(1, C), blk=1000)
    return out.reshape(B, N, C)
